# Initial kernel scaffold; baseline (speedup 1.0000x reference)
import jax
import jax.numpy as jnp
from jax.experimental import pallas as pl


def kernel(x):
    raise NotImplementedError("write your pallas kernel here")


# speedup vs baseline: 5.1179x; 5.1179x over previous
import jax
import jax.numpy as jnp
from jax.experimental import pallas as pl
from jax.experimental.pallas import tpu as pltpu

_N = 8192
_K = 4096
_ROWS = 128
_BLOCK_ROWS = 8


def _sdd_block(x_ref, o_ref):
    int_max = jnp.int32(2**31 - 1)
    int_min = jnp.int32(-(2**31))
    x = x_ref[...]
    xz = x + 0.0
    b = jax.lax.bitcast_convert_type(xz, jnp.int32)
    key = jnp.where(b >= 0, b, int_max - b)

    cnt_pos = jnp.sum((key >= 0).astype(jnp.int32), axis=1)
    prefix = jnp.where(cnt_pos >= _K, jnp.int32(0), int_min)

    def body(i, prefix):
        bit = jnp.left_shift(jnp.int32(1), jnp.int32(30) - i)
        cand = prefix + bit
        cnt = jnp.sum((key >= cand[:, None]).astype(jnp.int32), axis=1)
        return jnp.where(cnt >= _K, cand, prefix)

    prefix = jax.lax.fori_loop(0, 31, body, prefix)

    mask = key >= prefix[:, None]
    o_ref[...] = jnp.where(mask, x, jnp.float32(0.0))


def kernel(x):
    return pl.pallas_call(
        _sdd_block,
        out_shape=jax.ShapeDtypeStruct(x.shape, x.dtype),
        grid=(_ROWS // _BLOCK_ROWS,),
        in_specs=[pl.BlockSpec((_BLOCK_ROWS, _N), lambda i: (i, 0))],
        out_specs=pl.BlockSpec((_BLOCK_ROWS, _N), lambda i: (i, 0)),
        compiler_params=pltpu.CompilerParams(
            dimension_semantics=("parallel",)
        ),
    )(x)
